# baseline (device time: 182902 ns/iter reference)
import functools

import jax
import jax.numpy as jnp
from jax import lax
from jax.experimental import pallas as pl
from jax.experimental.pallas import tpu as pltpu

N_DEV = 32
N_TOK = 1024
D_MODEL = 512
D_OUT = 1024
E_LOCAL = 4
ROWS = N_TOK // N_DEV


def _ring_allreduce(partial_bf16):

    def body(p_ref, out_ref, send_buf, rs_buf, ag_buf,
             rs_send, rs_recv, ag_send, ag_recv):
        my = lax.axis_index("i")
        left = lax.rem(my + N_DEV - 1, N_DEV)
        right = lax.rem(my + 1, N_DEV)

        barrier_sem = pltpu.get_barrier_semaphore()
        for nbr in (left, right):
            pl.semaphore_signal(
                barrier_sem, inc=1,
                device_id=(nbr,), device_id_type=pl.DeviceIdType.MESH,
            )
        pl.semaphore_wait(barrier_sem, 2)

        send_buf[...] = p_ref[pl.ds(my * ROWS, ROWS), :]
        for h in range(N_DEV - 1):
            rdma = pltpu.make_async_remote_copy(
                src_ref=send_buf,
                dst_ref=rs_buf.at[h],
                send_sem=rs_send.at[h],
                recv_sem=rs_recv.at[h],
                device_id=(right,),
                device_id_type=pl.DeviceIdType.MESH,
            )
            rdma.start()
            rdma.wait()
            c = lax.rem(my + (N_DEV - h - 1), N_DEV)
            send_buf[...] = rs_buf[h] + p_ref[pl.ds(c * ROWS, ROWS), :]

        c_own = lax.rem(my + 1, N_DEV)
        out_ref[pl.ds(c_own * ROWS, ROWS), :] = send_buf[...].astype(jnp.float32)

        for h in range(N_DEV - 1):
            src = send_buf if h == 0 else ag_buf.at[h - 1]
            rdma = pltpu.make_async_remote_copy(
                src_ref=src,
                dst_ref=ag_buf.at[h],
                send_sem=ag_send.at[h],
                recv_sem=ag_recv.at[h],
                device_id=(right,),
                device_id_type=pl.DeviceIdType.MESH,
            )
            rdma.start()
            rdma.wait()
            c = lax.rem(my + (N_DEV - h), N_DEV)
            out_ref[pl.ds(c * ROWS, ROWS), :] = ag_buf[h].astype(jnp.float32)

        @functools.partial(
            pl.run_scoped, second_barrier=pltpu.SemaphoreType.REGULAR
        )
        def _(second_barrier):
            for nbr in (left, right):
                pl.semaphore_signal(
                    second_barrier, inc=1,
                    device_id=(nbr,), device_id_type=pl.DeviceIdType.MESH,
                )
            pl.semaphore_wait(second_barrier, 2)

    return pl.pallas_call(
        body,
        out_shape=jax.ShapeDtypeStruct((N_TOK, D_OUT), jnp.float32),
        in_specs=[pl.BlockSpec(memory_space=pltpu.VMEM)],
        out_specs=pl.BlockSpec(memory_space=pltpu.VMEM),
        scratch_shapes=[
            pltpu.VMEM((ROWS, D_OUT), jnp.bfloat16),
            pltpu.VMEM((N_DEV - 1, ROWS, D_OUT), jnp.bfloat16),
            pltpu.VMEM((N_DEV - 1, ROWS, D_OUT), jnp.bfloat16),
            pltpu.SemaphoreType.DMA((N_DEV - 1,)),
            pltpu.SemaphoreType.DMA((N_DEV - 1,)),
            pltpu.SemaphoreType.DMA((N_DEV - 1,)),
            pltpu.SemaphoreType.DMA((N_DEV - 1,)),
        ],
        compiler_params=pltpu.CompilerParams(collective_id=0),
    )(partial_bf16)


def kernel(x, router_W, route_idx, expert_W):
    my = lax.axis_index("i")

    scores = x @ router_W
    probs = jax.nn.softmax(scores, axis=-1)
    g = jnp.take_along_axis(probs, route_idx, axis=1)
    g = g / g.sum(axis=-1, keepdims=True)

    e_ids = my * E_LOCAL + jnp.arange(E_LOCAL)
    sel = route_idx[:, :, None] == e_ids[None, None, :]
    w = jnp.sum(g[:, :, None] * sel, axis=1)

    xw = (w[:, :, None] * x[:, None, :]).astype(jnp.bfloat16)
    xw2 = xw.reshape(N_TOK, E_LOCAL * D_MODEL)
    W2 = expert_W.astype(jnp.bfloat16).reshape(E_LOCAL * D_MODEL, D_OUT)
    partial = lax.dot(
        xw2, W2, preferred_element_type=jnp.float32
    ).astype(jnp.bfloat16)

    return _ring_allreduce(partial)


# device time: 88463 ns/iter; 2.0676x vs baseline; 2.0676x over previous
import functools

import jax
import jax.numpy as jnp
from jax import lax
from jax.experimental import pallas as pl
from jax.experimental.pallas import tpu as pltpu

N_DEV = 32
N_TOK = 1024
D_MODEL = 512
D_OUT = 1024
E_LOCAL = 4

MASKS = [3, 1, 8, 4, 16]


def kernel(x, router_W, route_idx, expert_W):
    def body(x_ref, rw_ref, idx_ref, ew_ref, out_ref, acc_ref,
             rs0, rs1, rs2, rs3, rs4, rs_send, rs_recv, ag_send, ag_recv):
        rs_bufs = [rs0, rs1, rs2, rs3, rs4]
        my = lax.axis_index("i")
        partners = [my ^ m for m in MASKS]

        barrier_sem = pltpu.get_barrier_semaphore()
        for p in partners:
            pl.semaphore_signal(
                barrier_sem, inc=1,
                device_id=(p,), device_id_type=pl.DeviceIdType.MESH,
            )
        pl.semaphore_wait(barrier_sem, len(partners))

        xv = x_ref[...]
        scores = jnp.dot(xv, rw_ref[...],
                         preferred_element_type=jnp.float32)
        idx0 = idx_ref[:, 0:1]
        idx1 = idx_ref[:, 1:2]
        e_iota = lax.broadcasted_iota(jnp.int32, (N_TOK, 128), 1)
        s0 = jnp.sum(jnp.where(e_iota == idx0, scores, 0.0), axis=1,
                     keepdims=True)
        s1 = jnp.sum(jnp.where(e_iota == idx1, scores, 0.0), axis=1,
                     keepdims=True)
        g0 = 1.0 / (1.0 + jnp.exp(s1 - s0))
        g1 = 1.0 - g0

        partial = jnp.zeros((N_TOK, D_OUT), jnp.float32)
        for j in range(E_LOCAL):
            e_j = my * E_LOCAL + j
            w_j = (jnp.where(idx0 == e_j, g0, 0.0)
                   + jnp.where(idx1 == e_j, g1, 0.0))
            xw = (xv * w_j).astype(jnp.bfloat16)
            partial += jnp.dot(
                xw, ew_ref[j].astype(jnp.bfloat16),
                preferred_element_type=jnp.float32,
            )
        acc_ref[...] = partial.astype(jnp.bfloat16)

        lo = jnp.int32(0)
        sz = N_TOK
        lo_hist = []
        for s, mask in enumerate(MASKS):
            half = sz // 2
            p = partners[s]
            i_am_low = my < p
            send_off = lo + jnp.where(i_am_low, half, 0)
            keep_off = lo + jnp.where(i_am_low, 0, half)
            rdma = pltpu.make_async_remote_copy(
                src_ref=acc_ref.at[pl.ds(send_off, half), :],
                dst_ref=rs_bufs[s],
                send_sem=rs_send.at[s],
                recv_sem=rs_recv.at[s],
                device_id=(p,),
                device_id_type=pl.DeviceIdType.MESH,
            )
            rdma.start()
            rdma.wait()
            acc_ref[pl.ds(keep_off, half), :] = (
                acc_ref[pl.ds(keep_off, half), :] + rs_bufs[s][...]
            )
            lo_hist.append(lo)
            lo = keep_off
            sz = half

        for jstep in range(len(MASKS)):
            s = len(MASKS) - 1 - jstep
            p = partners[s]
            cur_sz = N_TOK >> (s + 1)
            rdma = pltpu.make_async_remote_copy(
                src_ref=acc_ref.at[pl.ds(lo, cur_sz), :],
                dst_ref=acc_ref.at[pl.ds(lo, cur_sz), :],
                send_sem=ag_send.at[jstep],
                recv_sem=ag_recv.at[jstep],
                device_id=(p,),
                device_id_type=pl.DeviceIdType.MESH,
            )
            rdma.start()
            rdma.wait()
            lo = lo_hist[s]

        out_ref[...] = acc_ref[...].astype(jnp.float32)

        @functools.partial(
            pl.run_scoped, second_barrier=pltpu.SemaphoreType.REGULAR
        )
        def _(second_barrier):
            for p in partners:
                pl.semaphore_signal(
                    second_barrier, inc=1,
                    device_id=(p,), device_id_type=pl.DeviceIdType.MESH,
                )
            pl.semaphore_wait(second_barrier, len(partners))

    return pl.pallas_call(
        body,
        out_shape=jax.ShapeDtypeStruct((N_TOK, D_OUT), jnp.float32),
        in_specs=[
            pl.BlockSpec(memory_space=pltpu.VMEM),
            pl.BlockSpec(memory_space=pltpu.VMEM),
            pl.BlockSpec(memory_space=pltpu.VMEM),
            pl.BlockSpec(memory_space=pltpu.VMEM),
        ],
        out_specs=pl.BlockSpec(memory_space=pltpu.VMEM),
        scratch_shapes=[
            pltpu.VMEM((N_TOK, D_OUT), jnp.bfloat16),
            pltpu.VMEM((512, D_OUT), jnp.bfloat16),
            pltpu.VMEM((256, D_OUT), jnp.bfloat16),
            pltpu.VMEM((128, D_OUT), jnp.bfloat16),
            pltpu.VMEM((64, D_OUT), jnp.bfloat16),
            pltpu.VMEM((32, D_OUT), jnp.bfloat16),
            pltpu.SemaphoreType.DMA((len(MASKS),)),
            pltpu.SemaphoreType.DMA((len(MASKS),)),
            pltpu.SemaphoreType.DMA((len(MASKS),)),
            pltpu.SemaphoreType.DMA((len(MASKS),)),
        ],
        compiler_params=pltpu.CompilerParams(collective_id=0),
    )(x, router_W, route_idx, expert_W)


# device time: 67234 ns/iter; 2.7204x vs baseline; 1.3157x over previous
import functools

import jax
import jax.numpy as jnp
from jax import lax
from jax.experimental import pallas as pl
from jax.experimental.pallas import tpu as pltpu

N_DEV = 32
N_TOK = 1024
D_MODEL = 512
D_OUT = 1024
E_LOCAL = 4
COLS = D_OUT // 2

MASKS_A = [3, 1, 8, 4, 16]
MASKS_B = [8, 3, 1, 16, 4]
N_STEP = 5


def kernel(x, router_W, route_idx, expert_W):
    def body(x_ref, rw_ref, idx_ref, ew_ref, out_ref,
             rsA0, rsA1, rsA2, rsA3, rsA4,
             rsB0, rsB1, rsB2, rsB3, rsB4,
             rsA_send, rsA_recv, rsB_send, rsB_recv,
             agA_send, agA_recv, agB_send, agB_recv):
        my = lax.axis_index("i")
        all_partners = sorted({m for m in MASKS_A})
        assert all_partners == sorted({m for m in MASKS_B})

        barrier_sem = pltpu.get_barrier_semaphore()
        for m in all_partners:
            pl.semaphore_signal(
                barrier_sem, inc=1,
                device_id=(my ^ m,), device_id_type=pl.DeviceIdType.MESH,
            )
        pl.semaphore_wait(barrier_sem, len(all_partners))

        xv = x_ref[...]
        scores = jnp.dot(xv, rw_ref[...],
                         preferred_element_type=jnp.float32)
        idx0 = idx_ref[:, 0:1]
        idx1 = idx_ref[:, 1:2]
        e_iota = lax.broadcasted_iota(jnp.int32, (N_TOK, 128), 1)
        s0 = jnp.sum(jnp.where(e_iota == idx0, scores, 0.0), axis=1,
                     keepdims=True)
        s1 = jnp.sum(jnp.where(e_iota == idx1, scores, 0.0), axis=1,
                     keepdims=True)
        g0 = 1.0 / (1.0 + jnp.exp(s1 - s0))
        g1 = 1.0 - g0

        xw = []
        for j in range(E_LOCAL):
            e_j = my * E_LOCAL + j
            w_j = (jnp.where(idx0 == e_j, g0, 0.0)
                   + jnp.where(idx1 == e_j, g1, 0.0))
            xw.append((xv * w_j).astype(jnp.bfloat16))

        def compute_partial(col_lo):
            partial = jnp.zeros((N_TOK, COLS), jnp.float32)
            for j in range(E_LOCAL):
                wj = ew_ref[j, :, col_lo:col_lo + COLS].astype(jnp.bfloat16)
                partial += jnp.dot(xw[j], wj,
                                   preferred_element_type=jnp.float32)
            out_ref[:, col_lo:col_lo + COLS] = partial.astype(jnp.bfloat16)

        class Slice:
            pass

        def make_slice(masks, col_lo, rs_bufs, rs_send, rs_recv,
                       ag_send, ag_recv):
            st = Slice()
            st.masks = masks
            st.col_lo = col_lo
            st.rs_bufs = rs_bufs
            st.rs_send, st.rs_recv = rs_send, rs_recv
            st.ag_send, st.ag_recv = ag_send, ag_recv
            st.lo = jnp.int32(0)
            st.sz = N_TOK
            st.hist = []
            return st

        def rs_issue(st, s):
            half = st.sz // 2
            p = my ^ st.masks[s]
            low = my < p
            send_off = st.lo + jnp.where(low, half, 0)
            keep_off = st.lo + jnp.where(low, 0, half)
            rdma = pltpu.make_async_remote_copy(
                src_ref=out_ref.at[pl.ds(send_off, half),
                                   pl.ds(st.col_lo, COLS)],
                dst_ref=st.rs_bufs[s],
                send_sem=st.rs_send.at[s],
                recv_sem=st.rs_recv.at[s],
                device_id=(p,),
                device_id_type=pl.DeviceIdType.MESH,
            )
            rdma.start()
            st.pending = (rdma, keep_off, half, s)
            st.hist.append(st.lo)
            st.lo = keep_off
            st.sz = half

        def rs_finish(st):
            rdma, keep_off, half, s = st.pending
            rdma.wait()
            out_ref[pl.ds(keep_off, half), pl.ds(st.col_lo, COLS)] = (
                out_ref[pl.ds(keep_off, half), pl.ds(st.col_lo, COLS)]
                + st.rs_bufs[s][...]
            )

        def ag_issue(st, jstep):
            s = N_STEP - 1 - jstep
            p = my ^ st.masks[s]
            cur = N_TOK >> (s + 1)
            rdma = pltpu.make_async_remote_copy(
                src_ref=out_ref.at[pl.ds(st.lo, cur),
                                   pl.ds(st.col_lo, COLS)],
                dst_ref=out_ref.at[pl.ds(st.lo, cur),
                                   pl.ds(st.col_lo, COLS)],
                send_sem=st.ag_send.at[jstep],
                recv_sem=st.ag_recv.at[jstep],
                device_id=(p,),
                device_id_type=pl.DeviceIdType.MESH,
            )
            rdma.start()
            st.pending = rdma
            st.next_lo = st.hist[s]

        def ag_finish(st):
            st.pending.wait()
            st.lo = st.next_lo

        A = make_slice(MASKS_A, 0, [rsA0, rsA1, rsA2, rsA3, rsA4],
                       rsA_send, rsA_recv, agA_send, agA_recv)
        B = make_slice(MASKS_B, COLS, [rsB0, rsB1, rsB2, rsB3, rsB4],
                       rsB_send, rsB_recv, agB_send, agB_recv)

        compute_partial(A.col_lo)
        rs_issue(A, 0)
        compute_partial(B.col_lo)
        rs_issue(B, 0)
        for s in range(N_STEP):
            rs_finish(A)
            if s < N_STEP - 1:
                rs_issue(A, s + 1)
            rs_finish(B)
            if s < N_STEP - 1:
                rs_issue(B, s + 1)

        ag_issue(A, 0)
        ag_issue(B, 0)
        for jstep in range(N_STEP):
            ag_finish(A)
            if jstep < N_STEP - 1:
                ag_issue(A, jstep + 1)
            ag_finish(B)
            if jstep < N_STEP - 1:
                ag_issue(B, jstep + 1)

        @functools.partial(
            pl.run_scoped, second_barrier=pltpu.SemaphoreType.REGULAR
        )
        def _(second_barrier):
            for m in all_partners:
                pl.semaphore_signal(
                    second_barrier, inc=1,
                    device_id=(my ^ m,), device_id_type=pl.DeviceIdType.MESH,
                )
            pl.semaphore_wait(second_barrier, len(all_partners))

    dma = pltpu.SemaphoreType.DMA
    return pl.pallas_call(
        body,
        out_shape=jax.ShapeDtypeStruct((N_TOK, D_OUT), jnp.bfloat16),
        in_specs=[
            pl.BlockSpec(memory_space=pltpu.VMEM),
            pl.BlockSpec(memory_space=pltpu.VMEM),
            pl.BlockSpec(memory_space=pltpu.VMEM),
            pl.BlockSpec(memory_space=pltpu.VMEM),
        ],
        out_specs=pl.BlockSpec(memory_space=pltpu.VMEM),
        scratch_shapes=(
            [pltpu.VMEM((N_TOK >> (s + 1), COLS), jnp.bfloat16)
             for s in range(N_STEP)]
            + [pltpu.VMEM((N_TOK >> (s + 1), COLS), jnp.bfloat16)
               for s in range(N_STEP)]
            + [dma((N_STEP,)) for _ in range(8)]
        ),
        compiler_params=pltpu.CompilerParams(collective_id=0),
    )(x, router_W, route_idx, expert_W)


# device time: 65505 ns/iter; 2.7922x vs baseline; 1.0264x over previous
import functools

import jax
import jax.numpy as jnp
from jax import lax
from jax.experimental import pallas as pl
from jax.experimental.pallas import tpu as pltpu

N_DEV = 32
N_TOK = 1024
D_MODEL = 512
D_OUT = 1024
E_LOCAL = 4
COLS = D_OUT // 2

MASKS_A = [3, 1, 8, 4, 16]
MASKS_B = [8, 3, 1, 16, 4]
N_STEP = 5


def kernel(x, router_W, route_idx, expert_W):
    def body(x_ref, rw_ref, idx_ref, ew_ref, out_ref,
             rsA0, rsA1, rsA2, rsA3, rsA4,
             rsB0, rsB1, rsB2, rsB3, rsB4,
             rsA_send, rsA_recv, rsB_send, rsB_recv,
             agA_send, agA_recv, agB_send, agB_recv):
        my = lax.axis_index("i")
        all_partners = sorted({m for m in MASKS_A})
        assert all_partners == sorted({m for m in MASKS_B})

        barrier_sem = pltpu.get_barrier_semaphore()
        for m in all_partners:
            pl.semaphore_signal(
                barrier_sem, inc=1,
                device_id=(my ^ m,), device_id_type=pl.DeviceIdType.MESH,
            )
        pl.semaphore_wait(barrier_sem, len(all_partners))

        xv = x_ref[...]
        scores = jnp.dot(xv, rw_ref[...],
                         preferred_element_type=jnp.float32)
        idx0 = idx_ref[:, 0:1]
        idx1 = idx_ref[:, 1:2]
        e_iota = lax.broadcasted_iota(jnp.int32, (N_TOK, 128), 1)
        s0 = jnp.sum(jnp.where(e_iota == idx0, scores, 0.0), axis=1,
                     keepdims=True)
        s1 = jnp.sum(jnp.where(e_iota == idx1, scores, 0.0), axis=1,
                     keepdims=True)
        g0 = 1.0 / (1.0 + jnp.exp(s1 - s0))
        g1 = 1.0 - g0

        xw = []
        for j in range(E_LOCAL):
            e_j = my * E_LOCAL + j
            w_j = (jnp.where(idx0 == e_j, g0, 0.0)
                   + jnp.where(idx1 == e_j, g1, 0.0))
            xw.append((xv * w_j).astype(jnp.bfloat16))

        def compute_partial(col_lo):
            partial = jnp.zeros((N_TOK, COLS), jnp.float32)
            for j in range(E_LOCAL):
                wj = ew_ref[j, :, col_lo:col_lo + COLS].astype(jnp.bfloat16)
                partial += jnp.dot(xw[j], wj,
                                   preferred_element_type=jnp.float32)
            out_ref[:, col_lo:col_lo + COLS] = partial.astype(jnp.bfloat16)

        class Slice:
            pass

        def make_slice(masks, col_lo, rs_bufs, rs_send, rs_recv,
                       ag_send, ag_recv):
            st = Slice()
            st.masks = masks
            st.col_lo = col_lo
            st.rs_bufs = rs_bufs
            st.rs_send, st.rs_recv = rs_send, rs_recv
            st.ag_send, st.ag_recv = ag_send, ag_recv
            st.lo = jnp.int32(0)
            st.sz = N_TOK
            st.hist = []
            return st

        def rs_issue(st, s):
            half = st.sz // 2
            p = my ^ st.masks[s]
            low = my < p
            send_off = st.lo + jnp.where(low, half, 0)
            keep_off = st.lo + jnp.where(low, 0, half)
            rdma = pltpu.make_async_remote_copy(
                src_ref=out_ref.at[pl.ds(send_off, half),
                                   pl.ds(st.col_lo, COLS)],
                dst_ref=st.rs_bufs[s],
                send_sem=st.rs_send.at[s],
                recv_sem=st.rs_recv.at[s],
                device_id=(p,),
                device_id_type=pl.DeviceIdType.MESH,
            )
            rdma.start()
            st.pending = (rdma, keep_off, half, s)
            st.hist.append(st.lo)
            st.lo = keep_off
            st.sz = half

        def add_sub(st, s, buf_base, off, n):
            out_ref[pl.ds(off, n), pl.ds(st.col_lo, COLS)] = (
                out_ref[pl.ds(off, n), pl.ds(st.col_lo, COLS)]
                + st.rs_bufs[s][pl.ds(off - buf_base, n), :]
            )

        def rs_step(st, s):
            rdma, keep_off, half, _ = st.pending
            rdma.wait()
            if s == N_STEP - 1:
                add_sub(st, s, keep_off, keep_off, half)
                ag_issue(st, 0)
                return
            half2 = half // 2
            p2 = my ^ st.masks[s + 1]
            low2 = my < p2
            send2 = keep_off + jnp.where(low2, half2, 0)
            keep2 = keep_off + jnp.where(low2, 0, half2)
            add_sub(st, s, keep_off, send2, half2)
            rs_issue(st, s + 1)
            add_sub(st, s, keep_off, keep2, half2)

        def ag_issue(st, jstep):
            s = N_STEP - 1 - jstep
            p = my ^ st.masks[s]
            cur = N_TOK >> (s + 1)
            rdma = pltpu.make_async_remote_copy(
                src_ref=out_ref.at[pl.ds(st.lo, cur),
                                   pl.ds(st.col_lo, COLS)],
                dst_ref=out_ref.at[pl.ds(st.lo, cur),
                                   pl.ds(st.col_lo, COLS)],
                send_sem=st.ag_send.at[jstep],
                recv_sem=st.ag_recv.at[jstep],
                device_id=(p,),
                device_id_type=pl.DeviceIdType.MESH,
            )
            rdma.start()
            st.pending = rdma
            st.next_lo = st.hist[s]

        def ag_finish(st):
            st.pending.wait()
            st.lo = st.next_lo

        A = make_slice(MASKS_A, 0, [rsA0, rsA1, rsA2, rsA3, rsA4],
                       rsA_send, rsA_recv, agA_send, agA_recv)
        B = make_slice(MASKS_B, COLS, [rsB0, rsB1, rsB2, rsB3, rsB4],
                       rsB_send, rsB_recv, agB_send, agB_recv)

        compute_partial(A.col_lo)
        rs_issue(A, 0)
        compute_partial(B.col_lo)
        rs_issue(B, 0)
        for s in range(N_STEP):
            rs_step(A, s)
            rs_step(B, s)

        for jstep in range(N_STEP):
            ag_finish(A)
            if jstep < N_STEP - 1:
                ag_issue(A, jstep + 1)
            ag_finish(B)
            if jstep < N_STEP - 1:
                ag_issue(B, jstep + 1)

        @functools.partial(
            pl.run_scoped, second_barrier=pltpu.SemaphoreType.REGULAR
        )
        def _(second_barrier):
            for m in all_partners:
                pl.semaphore_signal(
                    second_barrier, inc=1,
                    device_id=(my ^ m,), device_id_type=pl.DeviceIdType.MESH,
                )
            pl.semaphore_wait(second_barrier, len(all_partners))

    dma = pltpu.SemaphoreType.DMA
    return pl.pallas_call(
        body,
        out_shape=jax.ShapeDtypeStruct((N_TOK, D_OUT), jnp.bfloat16),
        in_specs=[
            pl.BlockSpec(memory_space=pltpu.VMEM),
            pl.BlockSpec(memory_space=pltpu.VMEM),
            pl.BlockSpec(memory_space=pltpu.VMEM),
            pl.BlockSpec(memory_space=pltpu.VMEM),
        ],
        out_specs=pl.BlockSpec(memory_space=pltpu.VMEM),
        scratch_shapes=(
            [pltpu.VMEM((N_TOK >> (s + 1), COLS), jnp.bfloat16)
             for s in range(N_STEP)]
            + [pltpu.VMEM((N_TOK >> (s + 1), COLS), jnp.bfloat16)
               for s in range(N_STEP)]
            + [dma((N_STEP,)) for _ in range(8)]
        ),
        compiler_params=pltpu.CompilerParams(collective_id=0),
    )(x, router_W, route_idx, expert_W)


# device time: 58899 ns/iter; 3.1053x vs baseline; 1.1122x over previous
import jax
import jax.numpy as jnp
from jax import lax
from jax.experimental import pallas as pl
from jax.experimental.pallas import tpu as pltpu

N_DEV = 32
N_TOK = 1024
D_MODEL = 512
D_OUT = 1024
E_LOCAL = 4
COLS = D_OUT // 2

MASKS_A = [3, 1, 8, 4, 16]
MASKS_B = [8, 3, 1, 16, 4]
N_STEP = 5


def kernel(x, router_W, route_idx, expert_W):
    def body(x_ref, rw_ref, idx_ref, ew_ref, out_ref,
             rsA0, rsA1, rsA2, rsA3, rsA4,
             rsB0, rsB1, rsB2, rsB3, rsB4,
             rsA_send, rsA_recv, rsB_send, rsB_recv,
             agA_send, agA_recv, agB_send, agB_recv):
        my = lax.axis_index("i")
        all_masks = sorted(set(MASKS_A))
        assert all_masks == sorted(set(MASKS_B))

        barrier_sem = pltpu.get_barrier_semaphore()
        for m in all_masks:
            pl.semaphore_signal(
                barrier_sem, inc=1,
                device_id=(my ^ m,), device_id_type=pl.DeviceIdType.MESH,
            )

        xv = x_ref[...]
        scores = jnp.dot(xv, rw_ref[...],
                         preferred_element_type=jnp.float32)
        idx0 = idx_ref[:, 0:1]
        idx1 = idx_ref[:, 1:2]
        e_iota = lax.broadcasted_iota(jnp.int32, (N_TOK, 128), 1)
        s0 = jnp.sum(jnp.where(e_iota == idx0, scores, 0.0), axis=1,
                     keepdims=True)
        s1 = jnp.sum(jnp.where(e_iota == idx1, scores, 0.0), axis=1,
                     keepdims=True)
        g0 = 1.0 / (1.0 + jnp.exp(s1 - s0))
        g1 = 1.0 - g0

        xw = []
        for j in range(E_LOCAL):
            e_j = my * E_LOCAL + j
            w_j = (jnp.where(idx0 == e_j, g0, 0.0)
                   + jnp.where(idx1 == e_j, g1, 0.0))
            xw.append((xv * w_j).astype(jnp.bfloat16))

        def compute_partial(col_lo):
            partial = jnp.zeros((N_TOK, COLS), jnp.float32)
            for j in range(E_LOCAL):
                wj = ew_ref[j, :, col_lo:col_lo + COLS].astype(jnp.bfloat16)
                partial += jnp.dot(xw[j], wj,
                                   preferred_element_type=jnp.float32)
            out_ref[:, col_lo:col_lo + COLS] = partial.astype(jnp.bfloat16)

        class Slice:
            pass

        def make_slice(masks, col_lo, rs_bufs, rs_send, rs_recv,
                       ag_send, ag_recv):
            st = Slice()
            st.masks = masks
            st.col_lo = col_lo
            st.rs_bufs = rs_bufs
            st.rs_send, st.rs_recv = rs_send, rs_recv
            st.ag_send, st.ag_recv = ag_send, ag_recv
            st.lo = jnp.int32(0)
            st.sz = N_TOK
            st.hist = []
            return st

        def xchg(st, off, n, sem_arr_pair, slot, p):
            send_arr, recv_arr = sem_arr_pair
            rdma = pltpu.make_async_remote_copy(
                src_ref=out_ref.at[pl.ds(off, n), pl.ds(st.col_lo, COLS)],
                dst_ref=out_ref.at[pl.ds(off, n), pl.ds(st.col_lo, COLS)],
                send_sem=send_arr.at[slot],
                recv_sem=recv_arr.at[slot],
                device_id=(p,),
                device_id_type=pl.DeviceIdType.MESH,
            )
            rdma.start()
            return rdma

        def rs_issue(st, s):
            half = st.sz // 2
            p = my ^ st.masks[s]
            low = my < p
            send_off = st.lo + jnp.where(low, half, 0)
            keep_off = st.lo + jnp.where(low, 0, half)
            rdma = pltpu.make_async_remote_copy(
                src_ref=out_ref.at[pl.ds(send_off, half),
                                   pl.ds(st.col_lo, COLS)],
                dst_ref=st.rs_bufs[s],
                send_sem=st.rs_send.at[s],
                recv_sem=st.rs_recv.at[s],
                device_id=(p,),
                device_id_type=pl.DeviceIdType.MESH,
            )
            rdma.start()
            st.pending = (rdma, keep_off, half, s)
            st.hist.append(st.lo)
            st.lo = keep_off
            st.sz = half

        def add_sub(st, s, buf_base, off, n):
            out_ref[pl.ds(off, n), pl.ds(st.col_lo, COLS)] = (
                out_ref[pl.ds(off, n), pl.ds(st.col_lo, COLS)]
                + st.rs_bufs[s][pl.ds(off - buf_base, n), :]
            )

        def rs_step(st, s):
            rdma, keep_off, half, _ = st.pending
            rdma.wait()
            if s == N_STEP - 1:
                add_sub(st, s, keep_off, keep_off, half)
                st.ag_pend = {}
                st.ag_pend[0] = [xchg(st, st.lo, OWN, st.ag_sems, 0,
                                      my ^ st.masks[N_STEP - 1])]
                st.ag_pend[1] = [xchg(st, st.lo, OWN, st.ag_sems, 1,
                                      my ^ st.masks[N_STEP - 2])]
                return
            half2 = half // 2
            p2 = my ^ st.masks[s + 1]
            low2 = my < p2
            send2 = keep_off + jnp.where(low2, half2, 0)
            keep2 = keep_off + jnp.where(low2, 0, half2)
            add_sub(st, s, keep_off, send2, half2)
            rs_issue(st, s + 1)
            add_sub(st, s, keep_off, keep2, half2)

        OWN = N_TOK >> N_STEP

        def ag_m_off(st, j):
            return st.lo if j < 0 else st.hist[4 - j]

        def ag_wait(st, level):
            for r in st.ag_pend[level]:
                r.wait()

        def ag_issue_next(st, j):
            p = my ^ st.masks[N_STEP - 1 - j]
            o_sz = OWN << (j - 1)
            o_off = ag_m_off(st, j - 2)
            n_off = 2 * ag_m_off(st, j - 1) + o_sz - o_off
            st.ag_pend.setdefault(j, []).append(
                xchg(st, n_off, o_sz, st.ag_sems, 4 + j, p))
            if j < N_STEP - 1:
                p_next = my ^ st.masks[N_STEP - 2 - j]
                st.ag_pend[j + 1] = [
                    xchg(st, ag_m_off(st, j - 1), o_sz * 2,
                         st.ag_sems, j + 1, p_next)]

        A = make_slice(MASKS_A, 0, [rsA0, rsA1, rsA2, rsA3, rsA4],
                       rsA_send, rsA_recv, agA_send, agA_recv)
        B = make_slice(MASKS_B, COLS, [rsB0, rsB1, rsB2, rsB3, rsB4],
                       rsB_send, rsB_recv, agB_send, agB_recv)
        A.ag_sems = (agA_send, agA_recv)
        B.ag_sems = (agB_send, agB_recv)

        compute_partial(A.col_lo)
        pl.semaphore_wait(barrier_sem, len(all_masks))
        rs_issue(A, 0)
        compute_partial(B.col_lo)
        rs_issue(B, 0)
        for s in range(N_STEP):
            rs_step(A, s)
            rs_step(B, s)

        for j in range(1, N_STEP):
            ag_wait(A, j - 1)
            ag_issue_next(A, j)
            ag_wait(B, j - 1)
            ag_issue_next(B, j)
        ag_wait(A, N_STEP - 1)
        ag_wait(B, N_STEP - 1)

    dma = pltpu.SemaphoreType.DMA
    return pl.pallas_call(
        body,
        out_shape=jax.ShapeDtypeStruct((N_TOK, D_OUT), jnp.bfloat16),
        in_specs=[
            pl.BlockSpec(memory_space=pltpu.VMEM),
            pl.BlockSpec(memory_space=pltpu.VMEM),
            pl.BlockSpec(memory_space=pltpu.VMEM),
            pl.BlockSpec(memory_space=pltpu.VMEM),
        ],
        out_specs=pl.BlockSpec(memory_space=pltpu.VMEM),
        scratch_shapes=(
            [pltpu.VMEM((N_TOK >> (s + 1), COLS), jnp.bfloat16)
             for s in range(N_STEP)]
            + [pltpu.VMEM((N_TOK >> (s + 1), COLS), jnp.bfloat16)
               for s in range(N_STEP)]
            + [dma((N_STEP,)) for _ in range(4)]
            + [dma((2 * N_STEP - 1,)) for _ in range(4)]
        ),
        compiler_params=pltpu.CompilerParams(collective_id=0),
    )(x, router_W, route_idx, expert_W)


# device time: 55805 ns/iter; 3.2775x vs baseline; 1.0554x over previous
import jax
import jax.numpy as jnp
from jax import lax
from jax.experimental import pallas as pl
from jax.experimental.pallas import tpu as pltpu

N_DEV = 32
N_TOK = 1024
D_MODEL = 512
D_OUT = 1024
E_LOCAL = 4
COLS = D_OUT // 2

MASKS_A = [3, 1, 8, 4, 16]
MASKS_B = [8, 3, 1, 16, 4]
N_STEP = 5


def kernel(x, router_W, route_idx, expert_W):
    def body(x_ref, rw_ref, idx_ref, ew_ref, out_ref,
             rsA0, rsA1, rsA2, rsA3, rsA4,
             rsB0, rsB1, rsB2, rsB3, rsB4,
             rsA_send, rsA_recv, rsB_send, rsB_recv,
             agA_send, agA_recv, agB_send, agB_recv):
        my = lax.axis_index("i")
        all_masks = sorted(set(MASKS_A))
        assert all_masks == sorted(set(MASKS_B))

        barrier_sem = pltpu.get_barrier_semaphore()
        for m in all_masks:
            pl.semaphore_signal(
                barrier_sem, inc=1,
                device_id=(my ^ m,), device_id_type=pl.DeviceIdType.MESH,
            )

        xv = x_ref[...]
        scores = jnp.dot(xv, rw_ref[...],
                         preferred_element_type=jnp.float32)
        idx0 = idx_ref[:, 0:1]
        idx1 = idx_ref[:, 1:2]
        e_iota = lax.broadcasted_iota(jnp.int32, (N_TOK, 128), 1)
        s0 = jnp.sum(jnp.where(e_iota == idx0, scores, 0.0), axis=1,
                     keepdims=True)
        s1 = jnp.sum(jnp.where(e_iota == idx1, scores, 0.0), axis=1,
                     keepdims=True)
        g0 = 1.0 / (1.0 + jnp.exp(s1 - s0))
        g1 = 1.0 - g0

        xw = []
        for j in range(E_LOCAL):
            e_j = my * E_LOCAL + j
            w_j = (jnp.where(idx0 == e_j, g0, 0.0)
                   + jnp.where(idx1 == e_j, g1, 0.0))
            xw.append((xv * w_j).astype(jnp.bfloat16))

        def compute_partial(col_lo):
            partial = jnp.zeros((N_TOK, COLS), jnp.float32)
            for j in range(E_LOCAL):
                wj = ew_ref[j, :, col_lo:col_lo + COLS].astype(jnp.bfloat16)
                partial += jnp.dot(xw[j], wj,
                                   preferred_element_type=jnp.float32)
            out_ref[:, col_lo:col_lo + COLS] = partial.astype(jnp.bfloat16)

        class Slice:
            pass

        def make_slice(masks, col_lo, rs_bufs, rs_send, rs_recv,
                       ag_send, ag_recv):
            st = Slice()
            st.masks = masks
            st.col_lo = col_lo
            st.rs_bufs = rs_bufs
            st.rs_send, st.rs_recv = rs_send, rs_recv
            st.ag_send, st.ag_recv = ag_send, ag_recv
            st.lo = jnp.int32(0)
            st.sz = N_TOK
            st.hist = []
            return st

        def xchg(st, off, n, sem_arr_pair, slot, p):
            send_arr, recv_arr = sem_arr_pair
            rdma = pltpu.make_async_remote_copy(
                src_ref=out_ref.at[pl.ds(off, n), pl.ds(st.col_lo, COLS)],
                dst_ref=out_ref.at[pl.ds(off, n), pl.ds(st.col_lo, COLS)],
                send_sem=send_arr.at[slot],
                recv_sem=recv_arr.at[slot],
                device_id=(p,),
                device_id_type=pl.DeviceIdType.MESH,
            )
            rdma.start()
            return rdma

        def rs_issue(st, s):
            half = st.sz // 2
            p = my ^ st.masks[s]
            low = my < p
            send_off = st.lo + jnp.where(low, half, 0)
            keep_off = st.lo + jnp.where(low, 0, half)

            def part(rel, n, slot):
                rdma = pltpu.make_async_remote_copy(
                    src_ref=out_ref.at[pl.ds(send_off + rel, n),
                                       pl.ds(st.col_lo, COLS)],
                    dst_ref=st.rs_bufs[s].at[pl.ds(rel, n), :],
                    send_sem=st.rs_send.at[slot],
                    recv_sem=st.rs_recv.at[slot],
                    device_id=(p,),
                    device_id_type=pl.DeviceIdType.MESH,
                )
                rdma.start()
                return rdma

            if s == N_STEP - 1:
                hot = part(jnp.int32(0), half, s)
                cold = None
            else:
                half2 = half // 2
                p_low2 = p < (p ^ st.masks[s + 1])
                rel_hot = jnp.where(p_low2, half2, 0)
                hot = part(rel_hot, half2, s)
                cold = part(half2 - rel_hot, half2, N_STEP + s)
            st.pending = (hot, cold, keep_off, half, s)
            st.hist.append(st.lo)
            st.lo = keep_off
            st.sz = half

        def add_sub(st, s, buf_base, off, n):
            out_ref[pl.ds(off, n), pl.ds(st.col_lo, COLS)] = (
                out_ref[pl.ds(off, n), pl.ds(st.col_lo, COLS)]
                + st.rs_bufs[s][pl.ds(off - buf_base, n), :]
            )

        def rs_step(st, s):
            hot, cold, keep_off, half, _ = st.pending
            hot.wait()
            if s == N_STEP - 1:
                add_sub(st, s, keep_off, keep_off, half)
                st.ag_pend = {}
                st.ag_pend[0] = [xchg(st, st.lo, OWN, st.ag_sems, 0,
                                      my ^ st.masks[N_STEP - 1])]
                st.ag_pend[1] = [xchg(st, st.lo, OWN, st.ag_sems, 1,
                                      my ^ st.masks[N_STEP - 2])]
                return
            half2 = half // 2
            p2 = my ^ st.masks[s + 1]
            low2 = my < p2
            send2 = keep_off + jnp.where(low2, half2, 0)
            keep2 = keep_off + jnp.where(low2, 0, half2)
            add_sub(st, s, keep_off, send2, half2)
            rs_issue(st, s + 1)
            cold.wait()
            add_sub(st, s, keep_off, keep2, half2)

        OWN = N_TOK >> N_STEP

        def ag_m_off(st, j):
            return st.lo if j < 0 else st.hist[4 - j]

        def ag_wait(st, level):
            for r in st.ag_pend[level]:
                r.wait()

        def ag_issue_next(st, j):
            p = my ^ st.masks[N_STEP - 1 - j]
            o_sz = OWN << (j - 1)
            o_off = ag_m_off(st, j - 2)
            n_off = 2 * ag_m_off(st, j - 1) + o_sz - o_off
            st.ag_pend.setdefault(j, []).append(
                xchg(st, n_off, o_sz, st.ag_sems, 4 + j, p))
            if j < N_STEP - 1:
                p_next = my ^ st.masks[N_STEP - 2 - j]
                st.ag_pend[j + 1] = [
                    xchg(st, ag_m_off(st, j - 1), o_sz * 2,
                         st.ag_sems, j + 1, p_next)]

        A = make_slice(MASKS_A, 0, [rsA0, rsA1, rsA2, rsA3, rsA4],
                       rsA_send, rsA_recv, agA_send, agA_recv)
        B = make_slice(MASKS_B, COLS, [rsB0, rsB1, rsB2, rsB3, rsB4],
                       rsB_send, rsB_recv, agB_send, agB_recv)
        A.ag_sems = (agA_send, agA_recv)
        B.ag_sems = (agB_send, agB_recv)

        compute_partial(A.col_lo)
        pl.semaphore_wait(barrier_sem, len(all_masks))
        rs_issue(A, 0)
        compute_partial(B.col_lo)
        rs_issue(B, 0)
        for s in range(N_STEP):
            rs_step(A, s)
            rs_step(B, s)

        for j in range(1, N_STEP):
            ag_wait(A, j - 1)
            ag_issue_next(A, j)
            ag_wait(B, j - 1)
            ag_issue_next(B, j)
        ag_wait(A, N_STEP - 1)
        ag_wait(B, N_STEP - 1)

    dma = pltpu.SemaphoreType.DMA
    return pl.pallas_call(
        body,
        out_shape=jax.ShapeDtypeStruct((N_TOK, D_OUT), jnp.bfloat16),
        in_specs=[
            pl.BlockSpec(memory_space=pltpu.VMEM),
            pl.BlockSpec(memory_space=pltpu.VMEM),
            pl.BlockSpec(memory_space=pltpu.VMEM),
            pl.BlockSpec(memory_space=pltpu.VMEM),
        ],
        out_specs=pl.BlockSpec(memory_space=pltpu.VMEM),
        scratch_shapes=(
            [pltpu.VMEM((N_TOK >> (s + 1), COLS), jnp.bfloat16)
             for s in range(N_STEP)]
            + [pltpu.VMEM((N_TOK >> (s + 1), COLS), jnp.bfloat16)
               for s in range(N_STEP)]
            + [dma((2 * N_STEP,)) for _ in range(4)]
            + [dma((2 * N_STEP - 1,)) for _ in range(4)]
        ),
        compiler_params=pltpu.CompilerParams(collective_id=0),
    )(x, router_W, route_idx, expert_W)
